# baseline (speedup 1.0000x reference)
import numpy as np
import jax
import jax.numpy as jnp
from jax import lax
from jax.experimental import pallas as pl
from jax.experimental.pallas import tpu as pltpu

_NMS_THR = 0.7
_STRIDE = 16
_N = 22500
_ROWS = 176
_COLS = 128
_NPAD = _ROWS * _COLS
_TOPN = 2000
_POST = 300
_OUTROWS = 304


def _anchors_np(H, W):
    base = 16.0
    ratios = np.array([0.5, 1.0, 2.0])
    scales = np.array([8.0, 16.0, 32.0])
    ws = np.round(np.sqrt(base * base / ratios))
    hs = np.round(ws * ratios)
    ws = (ws[:, None] * scales[None, :]).reshape(-1)
    hs = (hs[:, None] * scales[None, :]).reshape(-1)
    cx = (base - 1.0) / 2.0
    cy = (base - 1.0) / 2.0
    base_anchors = np.stack(
        [cx - 0.5 * (ws - 1), cy - 0.5 * (hs - 1),
         cx + 0.5 * (ws - 1), cy + 0.5 * (hs - 1)], axis=1)
    shift_x = np.arange(W) * _STRIDE
    shift_y = np.arange(H) * _STRIDE
    sx, sy = np.meshgrid(shift_x, shift_y)
    shifts = np.stack([sx.ravel(), sy.ravel(), sx.ravel(), sy.ravel()], axis=1)
    anchors = (shifts[:, None, :] + base_anchors[None, :, :]).reshape(-1, 4)
    return anchors.astype(np.float32)


def _body(info_s, cap_s, dat, anc,
          out_ref, x1r, y1r, x2r, y2r, arr, scr, ukr, eligr):
    dx, dy, dw, dh, sc = (dat.at[0], dat.at[1], dat.at[2], dat.at[3],
                          dat.at[4])
    aw, ah, acx, acy = anc.at[0], anc.at[1], anc.at[2], anc.at[3]
    img_h = info_s[0]
    img_w = info_s[1]
    msize = info_s[4] * jnp.maximum(info_s[2], info_s[3])

    awv = aw[...]
    ahv = ah[...]
    cx = dx[...] * awv + acx[...]
    cy = dy[...] * ahv + acy[...]
    pw = jnp.exp(dw[...]) * awv
    ph = jnp.exp(dh[...]) * ahv
    x1 = jnp.clip(cx - 0.5 * pw, 0.0, img_w - 1.0)
    y1 = jnp.clip(cy - 0.5 * ph, 0.0, img_h - 1.0)
    x2 = jnp.clip(cx + 0.5 * pw, 0.0, img_w - 1.0)
    y2 = jnp.clip(cy + 0.5 * ph, 0.0, img_h - 1.0)
    bw = x2 - x1 + 1.0
    bh = y2 - y1 + 1.0
    valid = (bw > msize) & (bh > msize)
    score = jnp.where(valid, sc[...], -1e9)

    rio = lax.broadcasted_iota(jnp.int32, (_ROWS, _COLS), 0)
    lio = lax.broadcasted_iota(jnp.int32, (_ROWS, _COLS), 1)
    flat = rio * _COLS + lio
    real = flat < _N

    bits = lax.bitcast_convert_type(score, jnp.int32)
    ukey = jnp.where(score >= 0.0, bits + 2, 1)
    ukey = jnp.where(real, ukey, 0)

    x1r[...] = x1
    y1r[...] = y1
    x2r[...] = x2
    y2r[...] = y2
    arr[...] = bw * bh
    scr[...] = score
    ukr[...] = ukey

    def bs_body(_, lohi):
        lo, hi = lohi
        mid = (lo + hi + 1) // 2
        cnt = jnp.sum((ukey >= mid).astype(jnp.int32))
        big = cnt >= _TOPN
        return (jnp.where(big, mid, lo), jnp.where(big, hi, mid - 1))

    lo, _ = lax.fori_loop(0, 30, bs_body,
                          (jnp.int32(0), jnp.int32((1 << 30) - 1)))
    thr_key = lo
    n_better = jnp.sum((ukey >= thr_key + 1).astype(jnp.int32))
    eq_needed = (_TOPN - n_better).astype(jnp.float32)

    eq = ukey == thr_key
    eqf = eq.astype(jnp.float32)
    ut = (lax.broadcasted_iota(jnp.int32, (_COLS, _COLS), 0)
          <= lax.broadcasted_iota(jnp.int32, (_COLS, _COLS), 1)
          ).astype(jnp.float32)
    rowcum = jnp.dot(eqf, ut, preferred_element_type=jnp.float32)
    rowtot = rowcum[:, _COLS - 1:_COLS]
    sl = (lax.broadcasted_iota(jnp.int32, (_ROWS, _ROWS), 1)
          < lax.broadcasted_iota(jnp.int32, (_ROWS, _ROWS), 0)
          ).astype(jnp.float32)
    rowoff = jnp.dot(sl, rowtot, preferred_element_type=jnp.float32)
    pref_excl = rowoff + rowcum - eqf
    keep = (ukey > thr_key) | (eq & (pref_excl < eq_needed))
    ukm0 = jnp.where(keep, ukey, -1)

    out_ref[...] = jnp.zeros((_OUTROWS, 8), jnp.float32)

    def cond(c):
        t, alive, _ = c
        return alive & (t < cap_s[0])

    def body(c):
        t, _, ukm = c
        m1 = jnp.max(ukm)
        alive = m1 >= 1
        r2 = lax.broadcasted_iota(jnp.int32, (_ROWS, _COLS), 0)
        l2 = lax.broadcasted_iota(jnp.int32, (_ROWS, _COLS), 1)
        fl = r2 * _COLS + l2
        pidx = jnp.min(jnp.where(ukm == m1, fl, jnp.int32(1 << 30)))
        pidx = jnp.minimum(pidx, _NPAD - 1)
        prow = pidx >> 7
        ohl = lax.broadcasted_iota(jnp.int32, (1, _COLS), 1) == (pidx & 127)
        zx1 = x1r[...]
        zy1 = y1r[...]
        zx2 = x2r[...]
        zy2 = y2r[...]
        za = arr[...]
        px1 = jnp.sum(jnp.where(ohl, x1r[pl.ds(prow, 1), :], 0.0))
        py1 = jnp.sum(jnp.where(ohl, y1r[pl.ds(prow, 1), :], 0.0))
        px2 = jnp.sum(jnp.where(ohl, x2r[pl.ds(prow, 1), :], 0.0))
        py2 = jnp.sum(jnp.where(ohl, y2r[pl.ds(prow, 1), :], 0.0))
        pa = jnp.sum(jnp.where(ohl, arr[pl.ds(prow, 1), :], 0.0))
        psc = jnp.sum(jnp.where(ohl, scr[pl.ds(prow, 1), :], 0.0))
        xx1 = jnp.maximum(zx1, px1)
        yy1 = jnp.maximum(zy1, py1)
        xx2 = jnp.minimum(zx2, px2)
        yy2 = jnp.minimum(zy2, py2)
        w = jnp.maximum(xx2 - xx1 + 1.0, 0.0)
        h = jnp.maximum(yy2 - yy1 + 1.0, 0.0)
        inter = w * h
        iou = inter / (pa + za - inter)
        new_ukm = jnp.where(alive & (iou > _NMS_THR), -1, ukm)

        @pl.when(alive)
        def _():
            li8 = lax.broadcasted_iota(jnp.int32, (1, 8), 1)
            row = jnp.where(
                li8 == 0, px1,
                jnp.where(li8 == 1, py1,
                          jnp.where(li8 == 2, px2,
                                    jnp.where(li8 == 3, py2,
                                              jnp.where(li8 == 4, psc, 0.0)))))
            out_ref[pl.ds(t, 1), :] = row

        return (t + 1, alive, new_ukm)

    lax.while_loop(cond, body, (jnp.int32(0), jnp.bool_(True), ukm0))


def kernel(cls_prob, loc_offset, im_info, min_size, topn, post_topn):
    B, C4, H, W = loc_offset.shape
    info = jnp.concatenate(
        [im_info.astype(jnp.float32),
         jnp.reshape(jnp.asarray(min_size, jnp.float32), (1,))])
    cap = jnp.reshape(
        jnp.minimum(jnp.asarray(post_topn, jnp.int32), _POST), (1,))

    anc = _anchors_np(H, W)
    aw = anc[:, 2] - anc[:, 0] + 1.0
    ah = anc[:, 3] - anc[:, 1] + 1.0
    acx = anc[:, 0] + 0.5 * aw
    acy = anc[:, 1] + 0.5 * ah

    def padgrid_np(v):
        return jnp.asarray(
            np.pad(v, (0, _NPAD - _N)).reshape(_ROWS, _COLS))

    def padgrid(v):
        return jnp.reshape(jnp.pad(v, (0, _NPAD - _N)), (_ROWS, _COLS))

    loc = jnp.transpose(loc_offset, (0, 2, 3, 1)).reshape(-1, 4)
    score = jnp.transpose(cls_prob, (0, 2, 3, 1)).reshape(-1)

    dat = jnp.stack([padgrid(loc[:, 0]), padgrid(loc[:, 1]),
                     padgrid(loc[:, 2]), padgrid(loc[:, 3]),
                     padgrid(score)])
    ancg = jnp.stack([padgrid_np(aw), padgrid_np(ah),
                      padgrid_np(acx), padgrid_np(acy)])
    res = pl.pallas_call(
        _body,
        in_specs=[pl.BlockSpec(memory_space=pltpu.SMEM)] * 2 +
                 [pl.BlockSpec(memory_space=pltpu.VMEM)] * 2,
        out_specs=pl.BlockSpec(memory_space=pltpu.VMEM),
        out_shape=jax.ShapeDtypeStruct((_OUTROWS, 8), jnp.float32),
        scratch_shapes=[pltpu.VMEM((_ROWS, _COLS), jnp.float32)] * 6 +
                       [pltpu.VMEM((_ROWS, _COLS), jnp.int32)] * 2,
    )(info, cap, dat, ancg)

    return res[:_POST, :4], res[:_POST, 4]


# speedup vs baseline: 1.0077x; 1.0077x over previous
import numpy as np
import jax
import jax.numpy as jnp
from jax import lax
from jax.experimental import pallas as pl
from jax.experimental.pallas import tpu as pltpu

_NMS_THR = 0.7
_STRIDE = 16
_N = 22500
_ROWS = 176
_COLS = 128
_NPAD = _ROWS * _COLS
_TOPN = 2000
_POST = 300
_OUTROWS = 304


def _anchors_np(H, W):
    base = 16.0
    ratios = np.array([0.5, 1.0, 2.0])
    scales = np.array([8.0, 16.0, 32.0])
    ws = np.round(np.sqrt(base * base / ratios))
    hs = np.round(ws * ratios)
    ws = (ws[:, None] * scales[None, :]).reshape(-1)
    hs = (hs[:, None] * scales[None, :]).reshape(-1)
    cx = (base - 1.0) / 2.0
    cy = (base - 1.0) / 2.0
    base_anchors = np.stack(
        [cx - 0.5 * (ws - 1), cy - 0.5 * (hs - 1),
         cx + 0.5 * (ws - 1), cy + 0.5 * (hs - 1)], axis=1)
    shift_x = np.arange(W) * _STRIDE
    shift_y = np.arange(H) * _STRIDE
    sx, sy = np.meshgrid(shift_x, shift_y)
    shifts = np.stack([sx.ravel(), sy.ravel(), sx.ravel(), sy.ravel()], axis=1)
    anchors = (shifts[:, None, :] + base_anchors[None, :, :]).reshape(-1, 4)
    return anchors.astype(np.float32)


def _body(info_s, cap_s, dat, anc,
          out_ref, x1r, y1r, x2r, y2r, arr, scr, ukr, eligr):
    dx, dy, dw, dh, sc = (dat.at[0], dat.at[1], dat.at[2], dat.at[3],
                          dat.at[4])
    aw, ah, acx, acy = anc.at[0], anc.at[1], anc.at[2], anc.at[3]
    img_h = info_s[0]
    img_w = info_s[1]
    msize = info_s[4] * jnp.maximum(info_s[2], info_s[3])

    awv = aw[...]
    ahv = ah[...]
    cx = dx[...] * awv + acx[...]
    cy = dy[...] * ahv + acy[...]
    pw = jnp.exp(dw[...]) * awv
    ph = jnp.exp(dh[...]) * ahv
    x1 = jnp.clip(cx - 0.5 * pw, 0.0, img_w - 1.0)
    y1 = jnp.clip(cy - 0.5 * ph, 0.0, img_h - 1.0)
    x2 = jnp.clip(cx + 0.5 * pw, 0.0, img_w - 1.0)
    y2 = jnp.clip(cy + 0.5 * ph, 0.0, img_h - 1.0)
    bw = x2 - x1 + 1.0
    bh = y2 - y1 + 1.0
    valid = (bw > msize) & (bh > msize)
    score = jnp.where(valid, sc[...], -1e9)

    rio = lax.broadcasted_iota(jnp.int32, (_ROWS, _COLS), 0)
    lio = lax.broadcasted_iota(jnp.int32, (_ROWS, _COLS), 1)
    flat = rio * _COLS + lio
    real = flat < _N

    bits = lax.bitcast_convert_type(score, jnp.int32)
    ukey = jnp.where(score >= 0.0, bits + 2, 1)
    ukey = jnp.where(real, ukey, 0)

    x1r[...] = x1
    y1r[...] = y1
    x2r[...] = x2
    y2r[...] = y2
    arr[...] = bw * bh
    scr[...] = score
    ukr[...] = ukey

    def bs_body(_, lohi):
        lo, hi = lohi
        mid = (lo + hi + 1) // 2
        cnt = jnp.sum((ukey >= mid).astype(jnp.int32))
        big = cnt >= _TOPN
        return (jnp.where(big, mid, lo), jnp.where(big, hi, mid - 1))

    lo, _ = lax.fori_loop(0, 30, bs_body,
                          (jnp.int32(0), jnp.int32((1 << 30) - 1)))
    thr_key = lo
    n_better = jnp.sum((ukey >= thr_key + 1).astype(jnp.int32))
    eq_needed = (_TOPN - n_better).astype(jnp.float32)

    eq = ukey == thr_key
    eqf = eq.astype(jnp.float32)
    ut = (lax.broadcasted_iota(jnp.int32, (_COLS, _COLS), 0)
          <= lax.broadcasted_iota(jnp.int32, (_COLS, _COLS), 1)
          ).astype(jnp.float32)
    rowcum = jnp.dot(eqf, ut, preferred_element_type=jnp.float32)
    rowtot = rowcum[:, _COLS - 1:_COLS]
    sl = (lax.broadcasted_iota(jnp.int32, (_ROWS, _ROWS), 1)
          < lax.broadcasted_iota(jnp.int32, (_ROWS, _ROWS), 0)
          ).astype(jnp.float32)
    rowoff = jnp.dot(sl, rowtot, preferred_element_type=jnp.float32)
    pref_excl = rowoff + rowcum - eqf
    keep = (ukey > thr_key) | (eq & (pref_excl < eq_needed))
    eligr[...] = jnp.where(keep, ukey, -1)


    out_ref[...] = jnp.zeros((_OUTROWS, 8), jnp.float32)

    def one_pivot(t, ukm):
        m1 = jnp.max(ukm)
        alive = m1 >= 1
        r2 = lax.broadcasted_iota(jnp.int32, (_ROWS, _COLS), 0)
        l2 = lax.broadcasted_iota(jnp.int32, (_ROWS, _COLS), 1)
        fl = r2 * _COLS + l2
        pidx = jnp.min(jnp.where(ukm == m1, fl, jnp.int32(1 << 30)))
        pidx = jnp.minimum(pidx, _NPAD - 1)
        prow = pidx >> 7
        ohl = lax.broadcasted_iota(jnp.int32, (1, _COLS), 1) == (pidx & 127)
        zx1 = x1r[...]
        zy1 = y1r[...]
        zx2 = x2r[...]
        zy2 = y2r[...]
        za = arr[...]
        px1 = jnp.sum(jnp.where(ohl, x1r[pl.ds(prow, 1), :], 0.0))
        py1 = jnp.sum(jnp.where(ohl, y1r[pl.ds(prow, 1), :], 0.0))
        px2 = jnp.sum(jnp.where(ohl, x2r[pl.ds(prow, 1), :], 0.0))
        py2 = jnp.sum(jnp.where(ohl, y2r[pl.ds(prow, 1), :], 0.0))
        pa = jnp.sum(jnp.where(ohl, arr[pl.ds(prow, 1), :], 0.0))
        psc = jnp.sum(jnp.where(ohl, scr[pl.ds(prow, 1), :], 0.0))
        xx1 = jnp.maximum(zx1, px1)
        yy1 = jnp.maximum(zy1, py1)
        xx2 = jnp.minimum(zx2, px2)
        yy2 = jnp.minimum(zy2, py2)
        w = jnp.maximum(xx2 - xx1 + 1.0, 0.0)
        h = jnp.maximum(yy2 - yy1 + 1.0, 0.0)
        inter = w * h
        iou = inter / (pa + za - inter)
        new_ukm = jnp.where(alive & (iou > _NMS_THR), -1, ukm)

        @pl.when(alive & (t < cap_s[0]))
        def _():
            li8 = lax.broadcasted_iota(jnp.int32, (1, 8), 1)
            row = jnp.where(
                li8 == 0, px1,
                jnp.where(li8 == 1, py1,
                          jnp.where(li8 == 2, px2,
                                    jnp.where(li8 == 3, py2,
                                              jnp.where(li8 == 4, psc, 0.0)))))
            out_ref[pl.ds(t, 1), :] = row

        return new_ukm, alive

    def cond(c):
        t, alive, _ = c
        return alive & (t < cap_s[0])

    def body(c):
        t, _, ukm = c
        ukm1, alive1 = one_pivot(t, ukm)
        ukm2, alive2 = one_pivot(t + 1, ukm1)
        return (t + 2, alive1 & alive2, ukm2)

    lax.while_loop(cond, body, (jnp.int32(0), jnp.bool_(True), eligr[...]))


def kernel(cls_prob, loc_offset, im_info, min_size, topn, post_topn):
    B, C4, H, W = loc_offset.shape
    info = jnp.concatenate(
        [im_info.astype(jnp.float32),
         jnp.reshape(jnp.asarray(min_size, jnp.float32), (1,))])
    cap = jnp.reshape(
        jnp.minimum(jnp.asarray(post_topn, jnp.int32), _POST), (1,))

    anc = _anchors_np(H, W)
    aw = anc[:, 2] - anc[:, 0] + 1.0
    ah = anc[:, 3] - anc[:, 1] + 1.0
    acx = anc[:, 0] + 0.5 * aw
    acy = anc[:, 1] + 0.5 * ah

    def padgrid_np(v):
        return jnp.asarray(
            np.pad(v, (0, _NPAD - _N)).reshape(_ROWS, _COLS))

    def padgrid(v):
        return jnp.reshape(jnp.pad(v, (0, _NPAD - _N)), (_ROWS, _COLS))

    loc = jnp.transpose(loc_offset, (0, 2, 3, 1)).reshape(-1, 4)
    score = jnp.transpose(cls_prob, (0, 2, 3, 1)).reshape(-1)

    dat = jnp.stack([padgrid(loc[:, 0]), padgrid(loc[:, 1]),
                     padgrid(loc[:, 2]), padgrid(loc[:, 3]),
                     padgrid(score)])
    ancg = jnp.stack([padgrid_np(aw), padgrid_np(ah),
                      padgrid_np(acx), padgrid_np(acy)])
    res = pl.pallas_call(
        _body,
        in_specs=[pl.BlockSpec(memory_space=pltpu.SMEM)] * 2 +
                 [pl.BlockSpec(memory_space=pltpu.VMEM)] * 2,
        out_specs=pl.BlockSpec(memory_space=pltpu.VMEM),
        out_shape=jax.ShapeDtypeStruct((_OUTROWS, 8), jnp.float32),
        scratch_shapes=[pltpu.VMEM((_ROWS, _COLS), jnp.float32)] * 6 +
                       [pltpu.VMEM((_ROWS, _COLS), jnp.int32)] * 2,
    )(info, cap, dat, ancg)

    return res[:_POST, :4], res[:_POST, 4]
